# baseline (speedup 1.0000x reference)
import functools

import jax
import jax.numpy as jnp
from jax import lax
from jax.experimental import pallas as pl
from jax.experimental.pallas import tpu as pltpu
from jax.experimental.pallas import tpu_sc as plsc

D = 768
DFF1 = 4 * D
DFF2 = 2 * D
E = 64
T = 2048
C = 40
SLOTS = E * C
OUT_ROWS = SLOTS + C
NC, NS = 2, 16
NW = NC * NS
RPW = T // NW


def _gelu(x):
    return 0.5 * x * (1.0 + lax.erf(x * 0.7071067811865476))


def _router_body(x_ref, wr_ref, slot_ref):
    logits = jnp.dot(x_ref[...], wr_ref[...],
                     preferred_element_type=jnp.float32)
    m = jnp.max(logits, axis=1, keepdims=True)
    col = lax.broadcasted_iota(jnp.int32, (T, E), 1)
    idx = jnp.min(jnp.where(logits >= m, col, E), axis=1, keepdims=True)
    one_hot = (col == idx).astype(jnp.float32)
    c = one_hot
    k = 1
    while k < T:
        c = c + jnp.concatenate(
            [jnp.zeros((k, E), jnp.float32), c[:-k]], axis=0)
        k *= 2
    pos_x = c - one_hot
    pos = jnp.sum(pos_x * one_hot, axis=1, keepdims=True).astype(jnp.int32)
    tid = lax.broadcasted_iota(jnp.int32, (T, 1), 0)
    slot = jnp.where(pos < C, idx * C + pos, SLOTS + (tid % NW))
    slot_ref[...] = slot


def _router(x, wr):
    return pl.pallas_call(
        _router_body,
        out_shape=jax.ShapeDtypeStruct((T, 1), jnp.int32),
    )(x, wr)


@functools.cache
def _sc_kernels():
    mesh = plsc.VectorSubcoreMesh(core_axis_name="c", subcore_axis_name="s")
    scratch = [
        pltpu.VMEM((RPW,), jnp.int32),
        pltpu.VMEM((RPW, D), jnp.float32),
        pltpu.SemaphoreType.DMA,
    ]

    @functools.partial(
        pl.kernel,
        mesh=mesh,
        out_type=jax.ShapeDtypeStruct((OUT_ROWS, D), jnp.float32),
        scratch_types=scratch,
    )
    def dispatch(x_hbm, slot_hbm, ein_hbm, idx_v, rows_v, sem):
        wid = lax.axis_index("s") * NC + lax.axis_index("c")
        base = wid * RPW
        pltpu.sync_copy(slot_hbm.at[pl.ds(base, RPW)], idx_v)
        pltpu.sync_copy(x_hbm.at[pl.ds(base, RPW)], rows_v)
        pltpu.async_copy(rows_v, ein_hbm.at[idx_v], sem).wait()

    @functools.partial(
        pl.kernel,
        mesh=mesh,
        out_type=jax.ShapeDtypeStruct((T, D), jnp.float32),
        scratch_types=scratch,
    )
    def combine(eout_hbm, slot_hbm, out_hbm, idx_v, rows_v, sem):
        wid = lax.axis_index("s") * NC + lax.axis_index("c")
        base = wid * RPW
        pltpu.sync_copy(slot_hbm.at[pl.ds(base, RPW)], idx_v)
        pltpu.async_copy(eout_hbm.at[idx_v], rows_v, sem).wait()
        pltpu.sync_copy(rows_v, out_hbm.at[pl.ds(base, RPW)])

    return dispatch, combine


FH = DFF1 // 2


def _ffn_body(x_ref, w1_ref, b1_ref, w2_ref, b2_ref, w3_ref, b3_ref,
              out_ref, acc_ref):
    e = pl.program_id(0)
    j = pl.program_id(1)

    @pl.when(e < E)
    def _():
        h1 = _gelu(
            jnp.dot(x_ref[...], w1_ref[0], preferred_element_type=jnp.float32)
            + b1_ref[0])
        part = jnp.dot(h1, w2_ref[0], preferred_element_type=jnp.float32)

        @pl.when(j == 0)
        def _():
            acc_ref[...] = part

        @pl.when(j == 1)
        def _():
            h2 = _gelu(acc_ref[...] + part + b2_ref[0])
            out_ref[...] = (
                jnp.dot(h2, w3_ref[0], preferred_element_type=jnp.float32)
                + b3_ref[0])

    @pl.when(jnp.logical_and(e == E, j == 0))
    def _():
        out_ref[...] = jnp.zeros((C, D), jnp.float32)


def _ffn(ein, w1, b1, w2, b2, w3, b3):
    cl = lambda e: jnp.minimum(e, E - 1)
    w3m = lambda e, j: (jnp.clip((2 * e + j - 1) // 2, 0, E - 1), 0, 0)
    return pl.pallas_call(
        _ffn_body,
        grid=(E + 1, 2),
        in_specs=[
            pl.BlockSpec((C, D), lambda e, j: (cl(e), 0)),
            pl.BlockSpec((1, D, FH), lambda e, j: (cl(e), 0, j)),
            pl.BlockSpec((1, 1, FH), lambda e, j: (cl(e), 0, j)),
            pl.BlockSpec((1, FH, DFF2), lambda e, j: (cl(e), j, 0)),
            pl.BlockSpec((1, 1, DFF2), lambda e, j: (cl(e), 0, 0)),
            pl.BlockSpec((1, DFF2, D), w3m),
            pl.BlockSpec((1, 1, D), lambda e, j: (cl(e), 0, 0)),
        ],
        out_specs=pl.BlockSpec((C, D), lambda e, j: (e, 0)),
        out_shape=jax.ShapeDtypeStruct((OUT_ROWS, D), jnp.float32),
        scratch_shapes=[pltpu.VMEM((C, DFF2), jnp.float32)],
    )(ein, w1, b1, w2, b2, w3, b3)


def kernel(hidden_states, W_router, W1, b1, W2, b2, W3, b3):
    B, S, _ = hidden_states.shape
    x = hidden_states.reshape(T, D)
    dispatch, combine = _sc_kernels()
    slot = _router(x, W_router).reshape(T)
    ein = dispatch(x, slot)
    eout = _ffn(ein, W1, b1.reshape(E, 1, DFF1), W2, b2.reshape(E, 1, DFF2),
                W3, b3.reshape(E, 1, D))
    out = combine(eout, slot)
    return out.reshape(B, S, D)


# speedup vs baseline: 1.0009x; 1.0009x over previous
import functools

import jax
import jax.numpy as jnp
from jax import lax
from jax.experimental import pallas as pl
from jax.experimental.pallas import tpu as pltpu
from jax.experimental.pallas import tpu_sc as plsc

D = 768
DFF1 = 4 * D
DFF2 = 2 * D
E = 64
T = 2048
C = 40
SLOTS = E * C
OUT_ROWS = SLOTS + C
NC, NS = 2, 16
NW = NC * NS
RPW = T // NW


def _gelu(x):
    return 0.5 * x * (1.0 + lax.erf(x * 0.7071067811865476))


def _router_body(x_ref, wr_ref, slot_ref):
    logits = jnp.dot(x_ref[...], wr_ref[...],
                     preferred_element_type=jnp.float32)
    m = jnp.max(logits, axis=1, keepdims=True)
    col = lax.broadcasted_iota(jnp.int32, (T, E), 1)
    idx = jnp.min(jnp.where(logits >= m, col, E), axis=1, keepdims=True)
    one_hot = (col == idx).astype(jnp.float32)
    c = one_hot
    k = 1
    while k < T:
        c = c + jnp.concatenate(
            [jnp.zeros((k, E), jnp.float32), c[:-k]], axis=0)
        k *= 2
    pos_x = c - one_hot
    pos = jnp.sum(pos_x * one_hot, axis=1, keepdims=True).astype(jnp.int32)
    tid = lax.broadcasted_iota(jnp.int32, (T, 1), 0)
    slot = jnp.where(pos < C, idx * C + pos, SLOTS + (tid % NW))
    slot_ref[...] = slot


def _router(x, wr):
    return pl.pallas_call(
        _router_body,
        out_shape=jax.ShapeDtypeStruct((T, 1), jnp.int32),
    )(x, wr)


@functools.cache
def _sc_kernels():
    mesh = plsc.VectorSubcoreMesh(core_axis_name="c", subcore_axis_name="s")
    scratch = [
        pltpu.VMEM((RPW,), jnp.int32),
        pltpu.VMEM((RPW, D), jnp.float32),
        pltpu.SemaphoreType.DMA,
    ]

    @functools.partial(
        pl.kernel,
        mesh=mesh,
        out_type=jax.ShapeDtypeStruct((OUT_ROWS, D), jnp.float32),
        scratch_types=scratch,
    )
    def dispatch(x_hbm, slot_hbm, ein_hbm, idx_v, rows_v, sem):
        wid = lax.axis_index("s") * NC + lax.axis_index("c")
        base = wid * RPW
        pltpu.sync_copy(slot_hbm.at[pl.ds(base, RPW)], idx_v)
        pltpu.sync_copy(x_hbm.at[pl.ds(base, RPW)], rows_v)
        pltpu.async_copy(rows_v, ein_hbm.at[idx_v], sem).wait()

    @functools.partial(
        pl.kernel,
        mesh=mesh,
        out_type=jax.ShapeDtypeStruct((T, D), jnp.float32),
        scratch_types=scratch,
    )
    def combine(eout_hbm, slot_hbm, out_hbm, idx_v, rows_v, sem):
        wid = lax.axis_index("s") * NC + lax.axis_index("c")
        base = wid * RPW
        pltpu.sync_copy(slot_hbm.at[pl.ds(base, RPW)], idx_v)
        pltpu.async_copy(eout_hbm.at[idx_v], rows_v, sem).wait()
        pltpu.sync_copy(rows_v, out_hbm.at[pl.ds(base, RPW)])

    return dispatch, combine


FH = DFF1 // 2


FQ = DFF1 // 4
H3 = DFF2 // 2


def _ffn_body(x_ref, w1a_ref, w1b_ref, b1_ref, w2a_ref, w2b_ref, b2_ref,
              w3a_ref, w3b_ref, b3_ref, out_ref, acc_ref):
    e = pl.program_id(0)
    j = pl.program_id(1)

    @pl.when(e < E)
    def _():
        x = x_ref[...]
        h1a = _gelu(
            jnp.dot(x, w1a_ref[0], preferred_element_type=jnp.float32)
            + b1_ref[0][:, :FQ])
        h1b = _gelu(
            jnp.dot(x, w1b_ref[0], preferred_element_type=jnp.float32)
            + b1_ref[0][:, FQ:])
        part = (jnp.dot(h1a, w2a_ref[0], preferred_element_type=jnp.float32)
                + jnp.dot(h1b, w2b_ref[0],
                          preferred_element_type=jnp.float32))

        @pl.when(j == 0)
        def _():
            acc_ref[...] = part

        @pl.when(j == 1)
        def _():
            h2 = _gelu(acc_ref[...] + part + b2_ref[0])
            out_ref[...] = (
                jnp.dot(h2[:, :H3], w3a_ref[0],
                        preferred_element_type=jnp.float32)
                + jnp.dot(h2[:, H3:], w3b_ref[0],
                          preferred_element_type=jnp.float32)
                + b3_ref[0])

    @pl.when(jnp.logical_and(e == E, j == 0))
    def _():
        out_ref[...] = jnp.zeros((C, D), jnp.float32)


def _ffn(ein, w1, b1, w2, b2, w3, b3):
    cl = lambda e: jnp.minimum(e, E - 1)
    w3am = lambda e, j: (jnp.clip((2 * e + j) // 2, 0, E - 1), 0, 0)
    w3bm = lambda e, j: (jnp.clip((2 * e + j - 1) // 2, 0, E - 1), 1, 0)
    return pl.pallas_call(
        _ffn_body,
        grid=(E + 1, 2),
        in_specs=[
            pl.BlockSpec((C, D), lambda e, j: (cl(e), 0)),
            pl.BlockSpec((1, D, FQ), lambda e, j: (cl(e), 0, 2 * j)),
            pl.BlockSpec((1, D, FQ), lambda e, j: (cl(e), 0, 2 * j + 1)),
            pl.BlockSpec((1, 1, FH), lambda e, j: (cl(e), 0, j)),
            pl.BlockSpec((1, FQ, DFF2), lambda e, j: (cl(e), 2 * j, 0)),
            pl.BlockSpec((1, FQ, DFF2), lambda e, j: (cl(e), 2 * j + 1, 0)),
            pl.BlockSpec((1, 1, DFF2), lambda e, j: (cl(e), 0, 0)),
            pl.BlockSpec((1, H3, D), w3am),
            pl.BlockSpec((1, H3, D), w3bm),
            pl.BlockSpec((1, 1, D), lambda e, j: (cl(e), 0, 0)),
        ],
        out_specs=pl.BlockSpec((C, D), lambda e, j: (e, 0)),
        out_shape=jax.ShapeDtypeStruct((OUT_ROWS, D), jnp.float32),
        scratch_shapes=[pltpu.VMEM((C, DFF2), jnp.float32)],
    )(ein, w1, w1, b1, w2, w2, b2, w3, w3, b3)


def kernel(hidden_states, W_router, W1, b1, W2, b2, W3, b3):
    B, S, _ = hidden_states.shape
    x = hidden_states.reshape(T, D)
    dispatch, combine = _sc_kernels()
    slot = _router(x, W_router).reshape(T)
    ein = dispatch(x, slot)
    eout = _ffn(ein, W1, b1.reshape(E, 1, DFF1), W2, b2.reshape(E, 1, DFF2),
                W3, b3.reshape(E, 1, D))
    out = combine(eout, slot)
    return out.reshape(B, S, D)
